# baseline (speedup 1.0000x reference)
import jax
import jax.numpy as jnp
from jax import lax
from jax.experimental import pallas as pl
from jax.experimental.pallas import tpu as pltpu
from jax.experimental.pallas import tpu_sc as plsc

NC, NS, L = 2, 16, 16
B = 16384
D = 128
N = 100000
TBL = 102400
ZPT = TBL // NS
ZTRASH = 2048
CPB = B // NS
RPW = B // (NC * NS)
KCH = RPW // 128
VMAXT = 6


def _rot(x, k, iota):
  idxs = ((iota + k) & (L - 1)).reshape(L, 1)
  dnums = lax.GatherDimensionNumbers(
      offset_dims=(), collapsed_slice_dims=(0,), start_index_map=(0,))
  return lax.gather(x, idxs, dnums, (1,),
                    mode=lax.GatherScatterMode.PROMISE_IN_BOUNDS)


def _body(idx_hbm, values_hbm, out_hbm,
          idx_v, jv, ones_v, zbuf, cbuf, sbuf, gbuf, ptrbuf, lbuf, t1buf,
          dupj, dupi, ibuf, jbuf,
          rows_v, t0, cnt, t1, sem, sem2):
  cid = lax.axis_index("c")
  sid = lax.axis_index("s")
  iota = lax.iota(jnp.int32, L)

  pltpu.sync_copy(idx_hbm.at[pl.ds(sid * 8, 8)], idx_v)
  ones_vec = jnp.full((L,), 1, jnp.int32)
  zero_vec = jnp.zeros((L,), jnp.int32)
  for v in range(8):
    ones_v[pl.ds(v * L, L)] = ones_vec
  for v in range(64):
    zbuf[pl.ds(v * L, L)] = zero_vec
  base = sid * CPB
  for r in range(8):
    for v in range(8):
      jv[r, pl.ds(v * L, L)] = base + r * 128 + v * L + iota
  zoff = sid * ZPT
  descs = []
  for q in range(6):
    descs.append(
        pltpu.async_copy(zbuf, cnt.at[pl.ds(zoff + q * 1024, 1024)], sem))
  descs.append(pltpu.async_copy(
      zbuf.at[pl.ds(0, 256)], cnt.at[pl.ds(zoff + 6144, 256)], sem))
  for d in descs:
    d.wait()
  plsc.subcore_barrier()

  descs = []
  for r in range(8):
    irow = idx_v.at[r]
    descs.append(pltpu.async_copy(ones_v, cnt.at[irow], sem, add=True))
    descs.append(pltpu.async_copy(jv.at[r], t0.at[irow], sem))
  for d in descs:
    d.wait()
  plsc.subcore_barrier()

  descs = []
  for r in range(8):
    irow = idx_v.at[r]
    descs.append(pltpu.async_copy(cnt.at[irow], cbuf.at[r], sem))
    descs.append(pltpu.async_copy(t0.at[irow], sbuf.at[r], sem))
  for d in descs:
    d.wait()
  cnt3 = jnp.int32(0)
  for r in range(8):
    for v in range(8):
      cvec = cbuf[r, pl.ds(v * L, L)]
      jvec = jv[r, pl.ds(v * L, L)]
      ivec = idx_v[r, pl.ds(v * L, L)]
      gvec = sbuf[r, pl.ds(v * L, L)]
      lose = jnp.logical_and(cvec == 2, gvec != jvec)
      pos = r * 128 + v * L + iota
      lbuf[r, pl.ds(v * L, L)] = jnp.where(
          lose, ivec, N + (pos & (ZTRASH - 1)))
      m = jnp.where(cvec >= 3, 1, 0)
      cs = plsc.cumsum(m)
      dest = jnp.where(m == 1, cnt3 + (cs - m), CPB + L + iota)
      plsc.store_scatter(dupj, [dest], jvec)
      plsc.store_scatter(dupi, [dest], ivec)
      cnt3 = cnt3 + cs[L - 1]
  descs = []
  for r in range(8):
    descs.append(pltpu.async_copy(jv.at[r], t1.at[lbuf.at[r]], sem))
  for d in descs:
    d.wait()
  plsc.subcore_barrier()

  for step in range(NS):

    @pl.when(sid == step)
    def _replay():
      for q in range(VMAXT):

        @pl.when(cnt3 > q * L)
        def _vreg():
          valid = iota < cnt3 - q * L
          jd = dupj[pl.ds(q * L, L)]
          idxd = dupi[pl.ds(q * L, L)]
          idxp = jnp.where(valid, idxd, -1 - iota)
          dead = jnp.logical_not(valid)
          for k in range(1, L):
            sh = _rot(idxp, k, iota)
            dead = jnp.logical_or(
                dead, jnp.logical_and(idxp == sh, iota < (L - k)))
          sidx = jnp.where(dead, N + q * L + iota, idxp)
          ibuf[pl.ds(0, L)] = sidx
          jbuf[pl.ds(0, L)] = jd
          pltpu.async_copy(jbuf, t0.at[ibuf], sem2).wait()

    plsc.subcore_barrier()

  wid = sid * NC + cid
  r0 = cid * 4
  descs = []
  for k in range(KCH):
    irow = idx_v.at[r0 + k]
    descs.append(pltpu.async_copy(t0.at[irow], gbuf.at[k], sem))
    descs.append(pltpu.async_copy(t1.at[irow], t1buf.at[k], sem))
  for d in descs:
    d.wait()
  for k in range(KCH):
    for v in range(8):
      c = cbuf[r0 + k, pl.ds(v * L, L)]
      graw = sbuf[r0 + k, pl.ds(v * L, L)]
      lj = t1buf[k, pl.ds(v * L, L)]
      g = gbuf[k, pl.ds(v * L, L)]
      two = jnp.maximum(graw, lj)
      ptrbuf[k, pl.ds(v * L, L)] = jnp.where(c == 2, two, g)
  obase = wid * RPW
  gd = [None] * KCH
  sd = [None] * KCH
  gd[0] = pltpu.async_copy(values_hbm.at[ptrbuf.at[0]], rows_v.at[0], sem)
  for k in range(KCH):
    if k >= 1:
      sd[k - 1].wait()
    if k + 1 < KCH:
      gd[k + 1] = pltpu.async_copy(
          values_hbm.at[ptrbuf.at[k + 1]], rows_v.at[(k + 1) % 2], sem)
    gd[k].wait()
    sd[k] = pltpu.async_copy(
        rows_v.at[k % 2], out_hbm.at[pl.ds(obase + k * 128, 128)], sem2)
  sd[KCH - 1].wait()


@jax.jit
def _dynamic_state(idx2d, values):
  mesh = plsc.VectorSubcoreMesh(
      core_axis_name="c", subcore_axis_name="s",
      num_cores=NC, num_subcores=NS)
  f = pl.kernel(
      _body,
      out_type=jax.ShapeDtypeStruct((B, D), jnp.float32),
      mesh=mesh,
      compiler_params=pltpu.CompilerParams(needs_layout_passes=False),
      scratch_types=[
          pltpu.VMEM((8, 128), jnp.int32),
          pltpu.VMEM((8, 128), jnp.int32),
          pltpu.VMEM((128,), jnp.int32),
          pltpu.VMEM((1024,), jnp.int32),
          pltpu.VMEM((8, 128), jnp.int32),
          pltpu.VMEM((8, 128), jnp.int32),
          pltpu.VMEM((KCH, 128), jnp.int32),
          pltpu.VMEM((KCH, 128), jnp.int32),
          pltpu.VMEM((8, 128), jnp.int32),
          pltpu.VMEM((KCH, 128), jnp.int32),
          pltpu.VMEM((CPB + 2 * L,), jnp.int32),
          pltpu.VMEM((CPB + 2 * L,), jnp.int32),
          pltpu.VMEM((L,), jnp.int32),
          pltpu.VMEM((L,), jnp.int32),
          pltpu.VMEM((2, 128, 128), jnp.float32),
          pltpu.VMEM_SHARED((TBL,), jnp.int32),
          pltpu.VMEM_SHARED((TBL,), jnp.int32),
          pltpu.VMEM_SHARED((TBL,), jnp.int32),
          pltpu.SemaphoreType.DMA,
          pltpu.SemaphoreType.DMA,
      ],
  )
  return f(idx2d, values)


def kernel(state_src, node_idxs, values):
  del state_src
  idx2d = node_idxs.reshape(128, 128)
  return _dynamic_state(idx2d, values)


# speedup vs baseline: 1.1720x; 1.1720x over previous
import jax
import jax.numpy as jnp
from jax import lax
from jax.experimental import pallas as pl
from jax.experimental.pallas import tpu as pltpu
from jax.experimental.pallas import tpu_sc as plsc

NC, NS, L = 2, 16, 16
B = 16384
D = 128
N = 100000
TBL = 102400
ZPT = TBL // NS
ZTRASH = 2048
CPB = B // NS
RPW = B // (NC * NS)
KCH = RPW // 128
VMAXT = 6


def _rot(x, k, iota):
  idxs = ((iota + k) & (L - 1)).reshape(L, 1)
  dnums = lax.GatherDimensionNumbers(
      offset_dims=(), collapsed_slice_dims=(0,), start_index_map=(0,))
  return lax.gather(x, idxs, dnums, (1,),
                    mode=lax.GatherScatterMode.PROMISE_IN_BOUNDS)


def _body(idx_hbm, values_hbm, out_hbm,
          idx_v, jv, ones_v, zbuf, cbuf, sbuf, gbuf, ptrbuf, lbuf, t1buf,
          dupj, dupi, ibuf, jbuf,
          rows_v, baton, t0, cnt, t1, sem, sem2):
  cid = lax.axis_index("c")
  sid = lax.axis_index("s")
  iota = lax.iota(jnp.int32, L)

  baton[0] = jnp.int32(0)
  pltpu.sync_copy(idx_hbm.at[pl.ds(sid * 8, 8)], idx_v)
  ones_vec = jnp.full((L,), 1, jnp.int32)
  zero_vec = jnp.zeros((L,), jnp.int32)
  for v in range(8):
    ones_v[pl.ds(v * L, L)] = ones_vec
  for v in range(64):
    zbuf[pl.ds(v * L, L)] = zero_vec
  base = sid * CPB
  for r in range(8):
    for v in range(8):
      jv[r, pl.ds(v * L, L)] = base + r * 128 + v * L + iota
  zoff = sid * ZPT
  descs = []
  for q in range(6):
    descs.append(
        pltpu.async_copy(zbuf, cnt.at[pl.ds(zoff + q * 1024, 1024)], sem))
  descs.append(pltpu.async_copy(
      zbuf.at[pl.ds(0, 256)], cnt.at[pl.ds(zoff + 6144, 256)], sem))
  for d in descs:
    d.wait()
  plsc.subcore_barrier()

  descs = []
  for r in range(8):
    irow = idx_v.at[r]
    descs.append(pltpu.async_copy(ones_v, cnt.at[irow], sem, add=True))
    descs.append(pltpu.async_copy(jv.at[r], t0.at[irow], sem))
  for d in descs:
    d.wait()
  plsc.subcore_barrier()

  descs = []
  for r in range(8):
    irow = idx_v.at[r]
    descs.append(pltpu.async_copy(cnt.at[irow], cbuf.at[r], sem))
    descs.append(pltpu.async_copy(t0.at[irow], sbuf.at[r], sem))
  for d in descs:
    d.wait()
  cnt3 = jnp.int32(0)
  for r in range(8):
    for v in range(8):
      cvec = cbuf[r, pl.ds(v * L, L)]
      jvec = jv[r, pl.ds(v * L, L)]
      ivec = idx_v[r, pl.ds(v * L, L)]
      gvec = sbuf[r, pl.ds(v * L, L)]
      lose = jnp.logical_and(cvec == 2, gvec != jvec)
      pos = r * 128 + v * L + iota
      lbuf[r, pl.ds(v * L, L)] = jnp.where(
          lose, ivec, N + (pos & (ZTRASH - 1)))
      m = jnp.where(cvec >= 3, 1, 0)
      cs = plsc.cumsum(m)
      dest = jnp.where(m == 1, cnt3 + (cs - m), CPB + L + iota)
      plsc.store_scatter(dupj, [dest], jvec)
      plsc.store_scatter(dupi, [dest], ivec)
      cnt3 = cnt3 + cs[L - 1]
  descs = []
  for r in range(8):
    descs.append(pltpu.async_copy(jv.at[r], t1.at[lbuf.at[r]], sem))
  for d in descs:
    d.wait()
  plsc.subcore_barrier()

  @pl.when(sid > 0)
  def _spin():
    def cond(v):
      return v == 0

    def poll(v):
      return plsc.fetch_and_add(baton.at[0], jnp.int32(0), subcore_id=sid)

    lax.while_loop(cond, poll, jnp.int32(0))

  for q in range(VMAXT):

    @pl.when(cnt3 > q * L)
    def _vreg():
      valid = iota < cnt3 - q * L
      jd = dupj[pl.ds(q * L, L)]
      idxd = dupi[pl.ds(q * L, L)]
      idxp = jnp.where(valid, idxd, -1 - iota)
      dead = jnp.logical_not(valid)
      for k in range(1, L):
        sh = _rot(idxp, k, iota)
        dead = jnp.logical_or(
            dead, jnp.logical_and(idxp == sh, iota < (L - k)))
      sidx = jnp.where(dead, N + q * L + iota, idxp)
      ibuf[pl.ds(0, L)] = sidx
      jbuf[pl.ds(0, L)] = jd
      pltpu.async_copy(jbuf, t0.at[ibuf], sem2).wait()

  @pl.when(sid < NS - 1)
  def _pass_baton():
    plsc.fetch_and_add(baton.at[0], jnp.int32(1), subcore_id=sid + 1)

  plsc.subcore_barrier()

  wid = sid * NC + cid
  r0 = cid * 4
  descs = []
  for k in range(KCH):
    irow = idx_v.at[r0 + k]
    descs.append(pltpu.async_copy(t0.at[irow], gbuf.at[k], sem))
    descs.append(pltpu.async_copy(t1.at[irow], t1buf.at[k], sem))
  for d in descs:
    d.wait()
  for k in range(KCH):
    for v in range(8):
      c = cbuf[r0 + k, pl.ds(v * L, L)]
      graw = sbuf[r0 + k, pl.ds(v * L, L)]
      lj = t1buf[k, pl.ds(v * L, L)]
      g = gbuf[k, pl.ds(v * L, L)]
      two = jnp.maximum(graw, lj)
      ptrbuf[k, pl.ds(v * L, L)] = jnp.where(c == 2, two, g)
  obase = wid * RPW
  gd = [None] * KCH
  sd = [None] * KCH
  gd[0] = pltpu.async_copy(values_hbm.at[ptrbuf.at[0]], rows_v.at[0], sem)
  for k in range(KCH):
    if k >= 1:
      sd[k - 1].wait()
    if k + 1 < KCH:
      gd[k + 1] = pltpu.async_copy(
          values_hbm.at[ptrbuf.at[k + 1]], rows_v.at[(k + 1) % 2], sem)
    gd[k].wait()
    sd[k] = pltpu.async_copy(
        rows_v.at[k % 2], out_hbm.at[pl.ds(obase + k * 128, 128)], sem2)
  sd[KCH - 1].wait()


@jax.jit
def _dynamic_state(idx2d, values):
  mesh = plsc.VectorSubcoreMesh(
      core_axis_name="c", subcore_axis_name="s",
      num_cores=NC, num_subcores=NS)
  f = pl.kernel(
      _body,
      out_type=jax.ShapeDtypeStruct((B, D), jnp.float32),
      mesh=mesh,
      compiler_params=pltpu.CompilerParams(needs_layout_passes=False),
      scratch_types=[
          pltpu.VMEM((8, 128), jnp.int32),
          pltpu.VMEM((8, 128), jnp.int32),
          pltpu.VMEM((128,), jnp.int32),
          pltpu.VMEM((1024,), jnp.int32),
          pltpu.VMEM((8, 128), jnp.int32),
          pltpu.VMEM((8, 128), jnp.int32),
          pltpu.VMEM((KCH, 128), jnp.int32),
          pltpu.VMEM((KCH, 128), jnp.int32),
          pltpu.VMEM((8, 128), jnp.int32),
          pltpu.VMEM((KCH, 128), jnp.int32),
          pltpu.VMEM((CPB + 2 * L,), jnp.int32),
          pltpu.VMEM((CPB + 2 * L,), jnp.int32),
          pltpu.VMEM((L,), jnp.int32),
          pltpu.VMEM((L,), jnp.int32),
          pltpu.VMEM((2, 128, 128), jnp.float32),
          pltpu.SMEM((1,), jnp.int32),
          pltpu.VMEM_SHARED((TBL,), jnp.int32),
          pltpu.VMEM_SHARED((TBL,), jnp.int32),
          pltpu.VMEM_SHARED((TBL,), jnp.int32),
          pltpu.SemaphoreType.DMA,
          pltpu.SemaphoreType.DMA,
      ],
  )
  return f(idx2d, values)


def kernel(state_src, node_idxs, values):
  del state_src
  idx2d = node_idxs.reshape(128, 128)
  return _dynamic_state(idx2d, values)
